# baseline (speedup 1.0000x reference)
import jax
import jax.numpy as jnp
from jax import lax
from jax.experimental import pallas as pl
from jax.experimental.pallas import tpu as pltpu
from jax.experimental.pallas import tpu_sc as plsc

_BN = 1000
_SUB = 80
_EPS = 400
_NSUB = _EPS // _SUB

_N = 10000
_E = 320000
_R = 6
_NR_PAD = 60160
_EPT = _E // 16


def _sc_mesh():
    return plsc.VectorSubcoreMesh(core_axis_name="c", subcore_axis_name="s")


def _zero_fill(ref, nvec):
    z = jnp.zeros((16,), jnp.float32)

    def body(i, _):
        ref[pl.ds(i * 16, 16)] = z
        return 0

    lax.fori_loop(0, nvec, body, 0)


def _zero_fill2d(ref):
    z = jnp.zeros((16,), jnp.float32)

    def body(i, _):
        for k in range(8):
            ref[i, pl.ds(k * 16, 16)] = z
        return 0

    lax.fori_loop(0, ref.shape[0], body, 0)


def _prep_body(esrc, edst, et, norm_out, gidx_out, cnt_sh, zbuf, ones,
               sbig, dbig, tbig, kbig, g0big, g1big, nbig, kidx_a, kidx_b,
               cbuf, ssem_a, ssem_b):
    c = lax.axis_index("c")
    s = lax.axis_index("s")

    @pl.when(c == 0)
    def _():
        _zero_fill(zbuf, 3760 // 16)
        pltpu.sync_copy(zbuf, cnt_sh.at[pl.ds(s * 3760, 3760)])

        def init_ones(i, _):
            ones[pl.ds(i * 16, 16)] = jnp.full((16,), 1.0, jnp.float32)
            return 0

        lax.fori_loop(0, _SUB // 16, init_ones, 0)
        plsc.subcore_barrier()

        e0 = s * _EPT

        def count_super(i, _):
            base = e0 + i * _EPS
            pltpu.sync_copy(esrc.at[pl.ds(base, _EPS)], sbig)
            pltpu.sync_copy(edst.at[pl.ds(base, _EPS)], dbig)
            pltpu.sync_copy(et.at[pl.ds(base, _EPS)], tbig)

            def vec(j, _):
                dv = dbig[pl.ds(j * 16, 16)]
                tv = tbig[pl.ds(j * 16, 16)]
                sv = sbig[pl.ds(j * 16, 16)]
                kbig[pl.ds(j * 16, 16)] = dv * _R + tv
                g0 = tv * _N + sv
                g0big[pl.ds(j * 16, 16)] = g0
                g1big[pl.ds(j * 16, 16)] = g0 + _R * _N
                return 0

            lax.fori_loop(0, _EPS // 16, vec, 0)
            pltpu.sync_copy(g0big, gidx_out.at[pl.ds(base, _EPS)])
            pltpu.sync_copy(g1big, gidx_out.at[pl.ds(_E + base, _EPS)])

            kbufs = (kidx_a, kidx_b)
            sems = (ssem_a, ssem_b)
            sdesc = [None] * _NSUB
            for j in range(_NSUB):
                kb = kbufs[j % 2]
                if j >= 2:
                    sdesc[j - 2].wait()
                for k in range(_SUB // 16):
                    kb[pl.ds(k * 16, 16)] = kbig[pl.ds(j * _SUB + k * 16, 16)]
                sdesc[j] = pltpu.async_copy(ones, cnt_sh.at[kb], sems[j % 2],
                                            add=True)
            sdesc[_NSUB - 2].wait()
            sdesc[_NSUB - 1].wait()
            return 0

        lax.fori_loop(0, _EPT // _EPS, count_super, 0)
        plsc.subcore_barrier()

        pltpu.sync_copy(cnt_sh, cbuf)

        def norm_super(i, _):
            base = e0 + i * _EPS
            pltpu.sync_copy(edst.at[pl.ds(base, _EPS)], dbig)
            pltpu.sync_copy(et.at[pl.ds(base, _EPS)], tbig)

            def vec(j, _):
                dv = dbig[pl.ds(j * 16, 16)]
                tv = tbig[pl.ds(j * 16, 16)]
                cv = plsc.load_gather(cbuf, [dv * _R + tv])
                nbig[pl.ds(j * 16, 16)] = 1.0 / jnp.maximum(cv, 1.0)
                return 0

            lax.fori_loop(0, _EPS // 16, vec, 0)
            pltpu.sync_copy(nbig, norm_out.at[pl.ds(base, _EPS)])
            return 0

        lax.fori_loop(0, _EPT // _EPS, norm_super, 0)


def _sc_prep(esrc, edst, edge_types):
    f = pl.kernel(
        _prep_body,
        out_type=(
            jax.ShapeDtypeStruct((_E,), jnp.float32),
            jax.ShapeDtypeStruct((2 * _E,), jnp.int32),
        ),
        mesh=_sc_mesh(),
        scratch_types=[
            pltpu.MemorySpace.VMEM_SHARED((_NR_PAD,), jnp.float32),
            pltpu.VMEM((3760,), jnp.float32),
            pltpu.VMEM((_SUB,), jnp.float32),
            pltpu.VMEM((_EPS,), jnp.int32),
            pltpu.VMEM((_EPS,), jnp.int32),
            pltpu.VMEM((_EPS,), jnp.int32),
            pltpu.VMEM((_EPS,), jnp.int32),
            pltpu.VMEM((_EPS,), jnp.int32),
            pltpu.VMEM((_EPS,), jnp.int32),
            pltpu.VMEM((_EPS,), jnp.float32),
            pltpu.VMEM((_SUB,), jnp.int32),
            pltpu.VMEM((_SUB,), jnp.int32),
            pltpu.VMEM((_NR_PAD,), jnp.float32),
            pltpu.SemaphoreType.DMA,
            pltpu.SemaphoreType.DMA,
        ],
        compiler_params=pltpu.CompilerParams(needs_layout_passes=False),
    )
    return f(esrc, edst, edge_types)


def _agg_body(xw, gidx2, edst, norm, out, agg_sh, zbuf, gbig, dbig, nbig,
              rows0, rows1, rows2, didx0, didx1, didx2,
              gidx0, gidx1, gidx2b, gsem0, gsem1, gsem2,
              ssem0, ssem1, ssem2):
    c = lax.axis_index("c")
    s = lax.axis_index("s")

    _zero_fill2d(zbuf)
    z0 = s * 624
    for k in range(4):
        pltpu.sync_copy(zbuf, agg_sh.at[pl.ds(z0 + k * 128, 128)])
    pltpu.sync_copy(zbuf.at[pl.ds(0, 112)], agg_sh.at[pl.ds(z0 + 512, 112)])

    @pl.when(s == 0)
    def _():
        pltpu.sync_copy(zbuf.at[pl.ds(0, 16)], agg_sh.at[pl.ds(9984, 16)])

    plsc.subcore_barrier()

    e0 = s * _EPT
    rows = (rows0, rows1, rows2)
    didx = (didx0, didx1, didx2)
    gidx = (gidx0, gidx1, gidx2b)
    gsem = (gsem0, gsem1, gsem2)
    ssem = (ssem0, ssem1, ssem2)

    def scale(rbuf, joff):
        base = jnp.full((16,), joff, jnp.int32)

        def body(t, _):
            e = t * 2
            for d in range(2):
                ns = plsc.load_gather(nbig, [base + (e + d)])
                for k in range(8):
                    rbuf[e + d, pl.ds(k * 16, 16)] = (
                        rbuf[e + d, pl.ds(k * 16, 16)] * ns)
            return 0

        lax.fori_loop(0, _SUB // 2, body, 0)

    def fill_idx(dst_ref, src_ref, joff):
        for k in range(_SUB // 16):
            dst_ref[pl.ds(k * 16, 16)] = src_ref[pl.ds(joff + k * 16, 16)]

    def super_chunk(i, _):
        base = e0 + i * _EPS
        pltpu.sync_copy(gidx2.at[pl.ds(c * _E + base, _EPS)], gbig)
        pltpu.sync_copy(edst.at[pl.ds(base, _EPS)], dbig)
        pltpu.sync_copy(norm.at[pl.ds(base, _EPS)], nbig)

        gdesc = [None] * _NSUB
        sdesc = [None] * _NSUB
        for p in range(2):
            b = p % 3
            fill_idx(gidx[b], gbig, p * _SUB)
            gdesc[p] = pltpu.async_copy(xw.at[gidx[b]], rows[b], gsem[b])
        for j in range(_NSUB):
            b = j % 3
            gdesc[j].wait()
            fill_idx(didx[b], dbig, j * _SUB)
            scale(rows[b], j * _SUB)
            sdesc[j] = pltpu.async_copy(rows[b], agg_sh.at[didx[b]], ssem[b],
                                        add=True)
            jn = j + 2
            if jn < _NSUB:
                bn = jn % 3
                if jn - 3 >= 0:
                    sdesc[jn - 3].wait()
                fill_idx(gidx[bn], gbig, jn * _SUB)
                gdesc[jn] = pltpu.async_copy(xw.at[gidx[bn]], rows[bn],
                                             gsem[bn])
        for j in range(max(0, _NSUB - 3), _NSUB):
            sdesc[j].wait()
        return 0

    lax.fori_loop(0, _EPT // _EPS, super_chunk, 0)
    plsc.subcore_barrier()

    @pl.when(s == 0)
    def _():
        pltpu.sync_copy(agg_sh, out.at[c])


def _sc_aggregate(xw2, gidx2, edst, norm):
    f = pl.kernel(
        _agg_body,
        out_type=jax.ShapeDtypeStruct((2, _N, 128), jnp.float32),
        mesh=_sc_mesh(),
        scratch_types=(
            [pltpu.MemorySpace.VMEM_SHARED((_N, 128), jnp.float32),
             pltpu.VMEM((128, 128), jnp.float32),
             pltpu.VMEM((_EPS,), jnp.int32),
             pltpu.VMEM((_EPS,), jnp.int32),
             pltpu.VMEM((_EPS,), jnp.float32)]
            + [pltpu.VMEM((_SUB, 128), jnp.float32)] * 3
            + [pltpu.VMEM((_SUB,), jnp.int32)] * 3
            + [pltpu.VMEM((_SUB,), jnp.int32)] * 3
            + [pltpu.SemaphoreType.DMA] * 6
        ),
        compiler_params=pltpu.CompilerParams(needs_layout_passes=False),
    )
    return f(xw2, gidx2, edst, norm)


def _inproj_body(h_ref, w_ref, b_ref, o_ref):
    y = jnp.dot(h_ref[...], w_ref[...], preferred_element_type=jnp.float32)
    o_ref[...] = jax.nn.relu(y + b_ref[...])


def _input_proj(h_text, W_in, b_in):
    n, hid = h_text.shape
    d = W_in.shape[1]
    return pl.pallas_call(
        _inproj_body,
        grid=(n // _BN,),
        in_specs=[
            pl.BlockSpec((_BN, hid), lambda i: (i, 0)),
            pl.BlockSpec((hid, d), lambda i: (0, 0)),
            pl.BlockSpec((1, d), lambda i: (0, 0)),
        ],
        out_specs=pl.BlockSpec((_BN, d), lambda i: (i, 0)),
        out_shape=jax.ShapeDtypeStruct((n, d), jnp.float32),
    )(h_text, W_in, b_in.reshape(1, d))


def _wcomb_body(comp_ref, bases_ref, o_ref):
    r, b = comp_ref.shape[1], comp_ref.shape[2]
    for ri in range(r):
        acc = comp_ref[0, ri, 0] * bases_ref[0, 0]
        for i in range(1, b):
            acc = acc + comp_ref[0, ri, i] * bases_ref[0, i]
        o_ref[0, ri] = acc


def _basis_combine(comp, bases):
    ll, r, b = comp.shape
    d = bases.shape[-1]
    return pl.pallas_call(
        _wcomb_body,
        grid=(ll,),
        in_specs=[
            pl.BlockSpec((1, r, b), lambda l: (l, 0, 0)),
            pl.BlockSpec((1, b, d, d), lambda l: (l, 0, 0, 0)),
        ],
        out_specs=pl.BlockSpec((1, r, d, d), lambda l: (l, 0, 0, 0)),
        out_shape=jax.ShapeDtypeStruct((ll, r, d, d), jnp.float32),
    )(comp, bases)


def _xw_body(x_ref, w_ref, o_ref):
    y = jnp.dot(x_ref[...], w_ref[0], preferred_element_type=jnp.float32)
    h = y.shape[-1] // 2
    o_ref[0, 0] = y[:, :h]
    o_ref[1, 0] = y[:, h:]


def _per_relation_matmul(x, W):
    n, d = x.shape
    r = W.shape[0]
    return pl.pallas_call(
        _xw_body,
        grid=(r, n // _BN),
        in_specs=[
            pl.BlockSpec((_BN, d), lambda ri, i: (i, 0)),
            pl.BlockSpec((1, d, d), lambda ri, i: (ri, 0, 0)),
        ],
        out_specs=pl.BlockSpec((2, 1, _BN, d // 2),
                               lambda ri, i: (0, ri, i, 0)),
        out_shape=jax.ShapeDtypeStruct((2, r, n, d // 2), jnp.float32),
    )(x, W)


def _post_body(x_ref, agg_ref, root_ref, rb_ref, g_ref, be_ref, o_ref):
    agg = jnp.concatenate([agg_ref[0], agg_ref[1]], axis=-1)
    y = agg + jnp.dot(x_ref[...], root_ref[...],
                      preferred_element_type=jnp.float32) + rb_ref[...]
    y = jax.nn.relu(y) + x_ref[...]
    scale = 1.0 / jnp.sqrt(1.0 + 1e-5)
    o_ref[...] = y * (g_ref[...] * scale) + be_ref[...]


def _layer_post(x, agg2, root, rbias, gamma, beta):
    n, d = x.shape
    return pl.pallas_call(
        _post_body,
        grid=(n // _BN,),
        in_specs=[
            pl.BlockSpec((_BN, d), lambda i: (i, 0)),
            pl.BlockSpec((2, _BN, d // 2), lambda i: (0, i, 0)),
            pl.BlockSpec((d, d), lambda i: (0, 0)),
            pl.BlockSpec((1, d), lambda i: (0, 0)),
            pl.BlockSpec((1, d), lambda i: (0, 0)),
            pl.BlockSpec((1, d), lambda i: (0, 0)),
        ],
        out_specs=pl.BlockSpec((_BN, d), lambda i: (i, 0)),
        out_shape=jax.ShapeDtypeStruct((n, d), jnp.float32),
    )(x, agg2, root, rbias.reshape(1, d), gamma.reshape(1, d),
      beta.reshape(1, d))


def _out_body(x_ref, w_ref, b_ref, g_ref, be_ref, o_ref):
    h = jnp.dot(x_ref[...], w_ref[...], preferred_element_type=jnp.float32)
    h = h + b_ref[...]
    mu = jnp.mean(h, axis=-1, keepdims=True)
    var = jnp.mean((h - mu) ** 2, axis=-1, keepdims=True)
    o_ref[...] = (h - mu) / jnp.sqrt(var + 1e-5) * g_ref[...] + be_ref[...]


def _output_proj(x, W_out, b_out, ln_gamma, ln_beta):
    n, d = x.shape
    hid = W_out.shape[1]
    return pl.pallas_call(
        _out_body,
        grid=(n // _BN,),
        in_specs=[
            pl.BlockSpec((_BN, d), lambda i: (i, 0)),
            pl.BlockSpec((d, hid), lambda i: (0, 0)),
            pl.BlockSpec((1, hid), lambda i: (0, 0)),
            pl.BlockSpec((1, hid), lambda i: (0, 0)),
            pl.BlockSpec((1, hid), lambda i: (0, 0)),
        ],
        out_specs=pl.BlockSpec((_BN, hid), lambda i: (i, 0)),
        out_shape=jax.ShapeDtypeStruct((n, hid), jnp.float32),
    )(x, W_out, b_out.reshape(1, hid), ln_gamma.reshape(1, hid),
      ln_beta.reshape(1, hid))


def kernel(h_text, edge_index, edge_types, W_in, b_in, bases, comp, root,
           rbias, bn_gamma, bn_beta, W_out, b_out, ln_gamma, ln_beta):
    num_l = comp.shape[0]

    x = _input_proj(h_text, W_in, b_in)
    W_all = _basis_combine(comp, bases)
    esrc = edge_index[0]
    edst = edge_index[1]
    norm, gidx2 = _sc_prep(esrc, edst, edge_types)

    for l in range(num_l):
        xw2 = _per_relation_matmul(x, W_all[l])
        xw2 = xw2.reshape(2 * _R * _N, 128)
        agg2 = _sc_aggregate(xw2, gidx2, edst, norm)
        x = _layer_post(x, agg2, root[l], rbias[l], bn_gamma[l], bn_beta[l])

    return _output_proj(x, W_out, b_out, ln_gamma, ln_beta)


# speedup vs baseline: 1.2311x; 1.2311x over previous
import jax
import jax.numpy as jnp
from jax import lax
from jax.experimental import pallas as pl
from jax.experimental.pallas import tpu as pltpu
from jax.experimental.pallas import tpu_sc as plsc

_BN = 1000
_SUB = 80
_EPS = 800
_NSUB = _EPS // _SUB

_N = 10000
_E = 320000
_R = 6
_NR_PAD = 60160
_EPT = _E // 16


def _sc_mesh():
    return plsc.VectorSubcoreMesh(core_axis_name="c", subcore_axis_name="s")


def _zero_fill(ref, nvec):
    z = jnp.zeros((16,), jnp.float32)

    def body(i, _):
        ref[pl.ds(i * 16, 16)] = z
        return 0

    lax.fori_loop(0, nvec, body, 0)


def _zero_fill2d(ref):
    z = jnp.zeros((16,), jnp.float32)

    def body(i, _):
        for k in range(8):
            ref[i, pl.ds(k * 16, 16)] = z
        return 0

    lax.fori_loop(0, ref.shape[0], body, 0)


def _prep_body(esrc, edst, et, norm_out, gidx_out, cnt_sh, zbuf, ones,
               sbig, dbig, tbig, kbig, g0big, g1big, nbig, kidx_a, kidx_b,
               cbuf, ssem_a, ssem_b):
    c = lax.axis_index("c")
    s = lax.axis_index("s")

    @pl.when(c == 0)
    def _():
        _zero_fill(zbuf, 3760 // 16)
        pltpu.sync_copy(zbuf, cnt_sh.at[pl.ds(s * 3760, 3760)])

        def init_ones(i, _):
            ones[pl.ds(i * 16, 16)] = jnp.full((16,), 1.0, jnp.float32)
            return 0

        lax.fori_loop(0, _SUB // 16, init_ones, 0)
        plsc.subcore_barrier()

        e0 = s * _EPT

        def count_super(i, _):
            base = e0 + i * _EPS
            pltpu.sync_copy(esrc.at[pl.ds(base, _EPS)], sbig)
            pltpu.sync_copy(edst.at[pl.ds(base, _EPS)], dbig)
            pltpu.sync_copy(et.at[pl.ds(base, _EPS)], tbig)

            def vec(j, _):
                dv = dbig[pl.ds(j * 16, 16)]
                tv = tbig[pl.ds(j * 16, 16)]
                sv = sbig[pl.ds(j * 16, 16)]
                kbig[pl.ds(j * 16, 16)] = dv * _R + tv
                g0 = tv * _N + sv
                g0big[pl.ds(j * 16, 16)] = g0
                g1big[pl.ds(j * 16, 16)] = g0 + _R * _N
                return 0

            lax.fori_loop(0, _EPS // 16, vec, 0)
            pltpu.sync_copy(g0big, gidx_out.at[pl.ds(base, _EPS)])
            pltpu.sync_copy(g1big, gidx_out.at[pl.ds(_E + base, _EPS)])

            kbufs = (kidx_a, kidx_b)
            sems = (ssem_a, ssem_b)
            sdesc = [None] * _NSUB
            for j in range(_NSUB):
                kb = kbufs[j % 2]
                if j >= 2:
                    sdesc[j - 2].wait()
                for k in range(_SUB // 16):
                    kb[pl.ds(k * 16, 16)] = kbig[pl.ds(j * _SUB + k * 16, 16)]
                sdesc[j] = pltpu.async_copy(ones, cnt_sh.at[kb], sems[j % 2],
                                            add=True)
            sdesc[_NSUB - 2].wait()
            sdesc[_NSUB - 1].wait()
            return 0

        lax.fori_loop(0, _EPT // _EPS, count_super, 0)
        plsc.subcore_barrier()

        pltpu.sync_copy(cnt_sh, cbuf)

        def norm_super(i, _):
            base = e0 + i * _EPS
            pltpu.sync_copy(edst.at[pl.ds(base, _EPS)], dbig)
            pltpu.sync_copy(et.at[pl.ds(base, _EPS)], tbig)

            def vec(j, _):
                dv = dbig[pl.ds(j * 16, 16)]
                tv = tbig[pl.ds(j * 16, 16)]
                cv = plsc.load_gather(cbuf, [dv * _R + tv])
                nbig[pl.ds(j * 16, 16)] = 1.0 / jnp.maximum(cv, 1.0)
                return 0

            lax.fori_loop(0, _EPS // 16, vec, 0)
            pltpu.sync_copy(nbig, norm_out.at[pl.ds(base, _EPS)])
            return 0

        lax.fori_loop(0, _EPT // _EPS, norm_super, 0)


def _sc_prep(esrc, edst, edge_types):
    f = pl.kernel(
        _prep_body,
        out_type=(
            jax.ShapeDtypeStruct((_E,), jnp.float32),
            jax.ShapeDtypeStruct((2 * _E,), jnp.int32),
        ),
        mesh=_sc_mesh(),
        scratch_types=[
            pltpu.MemorySpace.VMEM_SHARED((_NR_PAD,), jnp.float32),
            pltpu.VMEM((3760,), jnp.float32),
            pltpu.VMEM((_SUB,), jnp.float32),
            pltpu.VMEM((_EPS,), jnp.int32),
            pltpu.VMEM((_EPS,), jnp.int32),
            pltpu.VMEM((_EPS,), jnp.int32),
            pltpu.VMEM((_EPS,), jnp.int32),
            pltpu.VMEM((_EPS,), jnp.int32),
            pltpu.VMEM((_EPS,), jnp.int32),
            pltpu.VMEM((_EPS,), jnp.float32),
            pltpu.VMEM((_SUB,), jnp.int32),
            pltpu.VMEM((_SUB,), jnp.int32),
            pltpu.VMEM((_NR_PAD,), jnp.float32),
            pltpu.SemaphoreType.DMA,
            pltpu.SemaphoreType.DMA,
        ],
        compiler_params=pltpu.CompilerParams(needs_layout_passes=False),
    )
    return f(esrc, edst, edge_types)


def _agg_body(xw, gidx2, edst, norm, out, agg_sh, zbuf, gbig, dbig, nbig,
              rows0, rows1, rows2, didx0, didx1, didx2,
              gidx0, gidx1, gidx2b, gsem0, gsem1, gsem2,
              ssem0, ssem1, ssem2):
    c = lax.axis_index("c")
    s = lax.axis_index("s")

    _zero_fill2d(zbuf)
    z0 = s * 624
    for k in range(4):
        pltpu.sync_copy(zbuf, agg_sh.at[pl.ds(z0 + k * 128, 128)])
    pltpu.sync_copy(zbuf.at[pl.ds(0, 112)], agg_sh.at[pl.ds(z0 + 512, 112)])

    @pl.when(s == 0)
    def _():
        pltpu.sync_copy(zbuf.at[pl.ds(0, 16)], agg_sh.at[pl.ds(9984, 16)])

    plsc.subcore_barrier()

    e0 = s * _EPT
    rows = (rows0, rows1, rows2)
    didx = (didx0, didx1, didx2)
    gidx = (gidx0, gidx1, gidx2b)
    gsem = (gsem0, gsem1, gsem2)
    ssem = (ssem0, ssem1, ssem2)

    def scale(rbuf, joff):
        base = jnp.full((16,), joff, jnp.int32)

        def body(t, _):
            e = t * 2
            for d in range(2):
                ns = plsc.load_gather(nbig, [base + (e + d)])
                for k in range(8):
                    rbuf[e + d, pl.ds(k * 16, 16)] = (
                        rbuf[e + d, pl.ds(k * 16, 16)] * ns)
            return 0

        lax.fori_loop(0, _SUB // 2, body, 0)

    def fill_idx(dst_ref, src_ref, joff):
        for k in range(_SUB // 16):
            dst_ref[pl.ds(k * 16, 16)] = src_ref[pl.ds(joff + k * 16, 16)]

    def super_chunk(i, _):
        base = e0 + i * _EPS
        pltpu.sync_copy(gidx2.at[pl.ds(c * _E + base, _EPS)], gbig)
        pltpu.sync_copy(edst.at[pl.ds(base, _EPS)], dbig)
        pltpu.sync_copy(norm.at[pl.ds(base, _EPS)], nbig)

        gdesc = [None] * _NSUB
        sdesc = [None] * _NSUB
        for p in range(2):
            b = p % 3
            fill_idx(gidx[b], gbig, p * _SUB)
            gdesc[p] = pltpu.async_copy(xw.at[gidx[b]], rows[b], gsem[b])
        for j in range(_NSUB):
            b = j % 3
            gdesc[j].wait()
            fill_idx(didx[b], dbig, j * _SUB)
            scale(rows[b], j * _SUB)
            sdesc[j] = pltpu.async_copy(rows[b], agg_sh.at[didx[b]], ssem[b],
                                        add=True)
            jn = j + 2
            if jn < _NSUB:
                bn = jn % 3
                if jn - 3 >= 0:
                    sdesc[jn - 3].wait()
                fill_idx(gidx[bn], gbig, jn * _SUB)
                gdesc[jn] = pltpu.async_copy(xw.at[gidx[bn]], rows[bn],
                                             gsem[bn])
        for j in range(max(0, _NSUB - 3), _NSUB):
            sdesc[j].wait()
        return 0

    lax.fori_loop(0, _EPT // _EPS, super_chunk, 0)
    plsc.subcore_barrier()

    @pl.when(s == 0)
    def _():
        pltpu.sync_copy(agg_sh, out.at[c])


def _sc_aggregate(xw2, gidx2, edst, norm):
    f = pl.kernel(
        _agg_body,
        out_type=jax.ShapeDtypeStruct((2, _N, 128), jnp.float32),
        mesh=_sc_mesh(),
        scratch_types=(
            [pltpu.MemorySpace.VMEM_SHARED((_N, 128), jnp.float32),
             pltpu.VMEM((128, 128), jnp.float32),
             pltpu.VMEM((_EPS,), jnp.int32),
             pltpu.VMEM((_EPS,), jnp.int32),
             pltpu.VMEM((_EPS,), jnp.float32)]
            + [pltpu.VMEM((_SUB, 128), jnp.float32)] * 3
            + [pltpu.VMEM((_SUB,), jnp.int32)] * 3
            + [pltpu.VMEM((_SUB,), jnp.int32)] * 3
            + [pltpu.SemaphoreType.DMA] * 6
        ),
        compiler_params=pltpu.CompilerParams(needs_layout_passes=False),
    )
    return f(xw2, gidx2, edst, norm)


def _inproj_body(h_ref, w_ref, b_ref, o_ref):
    y = jnp.dot(h_ref[...], w_ref[...], preferred_element_type=jnp.float32)
    o_ref[...] = jax.nn.relu(y + b_ref[...])


def _input_proj(h_text, W_in, b_in):
    n, hid = h_text.shape
    d = W_in.shape[1]
    return pl.pallas_call(
        _inproj_body,
        grid=(n // _BN,),
        in_specs=[
            pl.BlockSpec((_BN, hid), lambda i: (i, 0)),
            pl.BlockSpec((hid, d), lambda i: (0, 0)),
            pl.BlockSpec((1, d), lambda i: (0, 0)),
        ],
        out_specs=pl.BlockSpec((_BN, d), lambda i: (i, 0)),
        out_shape=jax.ShapeDtypeStruct((n, d), jnp.float32),
    )(h_text, W_in, b_in.reshape(1, d))


def _wcomb_body(comp_ref, bases_ref, o_ref):
    r, b = comp_ref.shape[1], comp_ref.shape[2]
    for ri in range(r):
        acc = comp_ref[0, ri, 0] * bases_ref[0, 0]
        for i in range(1, b):
            acc = acc + comp_ref[0, ri, i] * bases_ref[0, i]
        o_ref[0, ri] = acc


def _basis_combine(comp, bases):
    ll, r, b = comp.shape
    d = bases.shape[-1]
    return pl.pallas_call(
        _wcomb_body,
        grid=(ll,),
        in_specs=[
            pl.BlockSpec((1, r, b), lambda l: (l, 0, 0)),
            pl.BlockSpec((1, b, d, d), lambda l: (l, 0, 0, 0)),
        ],
        out_specs=pl.BlockSpec((1, r, d, d), lambda l: (l, 0, 0, 0)),
        out_shape=jax.ShapeDtypeStruct((ll, r, d, d), jnp.float32),
    )(comp, bases)


def _xw_body(x_ref, w_ref, o_ref):
    y = jnp.dot(x_ref[...], w_ref[0], preferred_element_type=jnp.float32)
    h = y.shape[-1] // 2
    o_ref[0, 0] = y[:, :h]
    o_ref[1, 0] = y[:, h:]


def _per_relation_matmul(x, W):
    n, d = x.shape
    r = W.shape[0]
    return pl.pallas_call(
        _xw_body,
        grid=(r, n // _BN),
        in_specs=[
            pl.BlockSpec((_BN, d), lambda ri, i: (i, 0)),
            pl.BlockSpec((1, d, d), lambda ri, i: (ri, 0, 0)),
        ],
        out_specs=pl.BlockSpec((2, 1, _BN, d // 2),
                               lambda ri, i: (0, ri, i, 0)),
        out_shape=jax.ShapeDtypeStruct((2, r, n, d // 2), jnp.float32),
    )(x, W)


def _post_body(x_ref, agg_ref, root_ref, rb_ref, g_ref, be_ref, o_ref):
    agg = jnp.concatenate([agg_ref[0], agg_ref[1]], axis=-1)
    y = agg + jnp.dot(x_ref[...], root_ref[...],
                      preferred_element_type=jnp.float32) + rb_ref[...]
    y = jax.nn.relu(y) + x_ref[...]
    scale = 1.0 / jnp.sqrt(1.0 + 1e-5)
    o_ref[...] = y * (g_ref[...] * scale) + be_ref[...]


def _layer_post(x, agg2, root, rbias, gamma, beta):
    n, d = x.shape
    return pl.pallas_call(
        _post_body,
        grid=(n // _BN,),
        in_specs=[
            pl.BlockSpec((_BN, d), lambda i: (i, 0)),
            pl.BlockSpec((2, _BN, d // 2), lambda i: (0, i, 0)),
            pl.BlockSpec((d, d), lambda i: (0, 0)),
            pl.BlockSpec((1, d), lambda i: (0, 0)),
            pl.BlockSpec((1, d), lambda i: (0, 0)),
            pl.BlockSpec((1, d), lambda i: (0, 0)),
        ],
        out_specs=pl.BlockSpec((_BN, d), lambda i: (i, 0)),
        out_shape=jax.ShapeDtypeStruct((n, d), jnp.float32),
    )(x, agg2, root, rbias.reshape(1, d), gamma.reshape(1, d),
      beta.reshape(1, d))


def _out_body(x_ref, w_ref, b_ref, g_ref, be_ref, o_ref):
    h = jnp.dot(x_ref[...], w_ref[...], preferred_element_type=jnp.float32)
    h = h + b_ref[...]
    mu = jnp.mean(h, axis=-1, keepdims=True)
    var = jnp.mean((h - mu) ** 2, axis=-1, keepdims=True)
    o_ref[...] = (h - mu) / jnp.sqrt(var + 1e-5) * g_ref[...] + be_ref[...]


def _output_proj(x, W_out, b_out, ln_gamma, ln_beta):
    n, d = x.shape
    hid = W_out.shape[1]
    return pl.pallas_call(
        _out_body,
        grid=(n // _BN,),
        in_specs=[
            pl.BlockSpec((_BN, d), lambda i: (i, 0)),
            pl.BlockSpec((d, hid), lambda i: (0, 0)),
            pl.BlockSpec((1, hid), lambda i: (0, 0)),
            pl.BlockSpec((1, hid), lambda i: (0, 0)),
            pl.BlockSpec((1, hid), lambda i: (0, 0)),
        ],
        out_specs=pl.BlockSpec((_BN, hid), lambda i: (i, 0)),
        out_shape=jax.ShapeDtypeStruct((n, hid), jnp.float32),
    )(x, W_out, b_out.reshape(1, hid), ln_gamma.reshape(1, hid),
      ln_beta.reshape(1, hid))


def kernel(h_text, edge_index, edge_types, W_in, b_in, bases, comp, root,
           rbias, bn_gamma, bn_beta, W_out, b_out, ln_gamma, ln_beta):
    num_l = comp.shape[0]

    x = _input_proj(h_text, W_in, b_in)
    W_all = _basis_combine(comp, bases)
    esrc = edge_index[0]
    edst = edge_index[1]
    norm, gidx2 = _sc_prep(esrc, edst, edge_types)

    for l in range(num_l):
        xw2 = _per_relation_matmul(x, W_all[l])
        xw2 = xw2.reshape(2 * _R * _N, 128)
        agg2 = _sc_aggregate(xw2, gidx2, edst, norm)
        x = _layer_post(x, agg2, root[l], rbias[l], bn_gamma[l], bn_beta[l])

    return _output_proj(x, W_out, b_out, ln_gamma, ln_beta)


# speedup vs baseline: 1.2880x; 1.0462x over previous
import jax
import jax.numpy as jnp
from jax import lax
from jax.experimental import pallas as pl
from jax.experimental.pallas import tpu as pltpu
from jax.experimental.pallas import tpu_sc as plsc

_BN = 1000
_SUB = 80
_EPS = 800
_NSUB = _EPS // _SUB

_N = 10000
_E = 320000
_R = 6
_NR_PAD = 60160
_EPT = _E // 16


def _sc_mesh():
    return plsc.VectorSubcoreMesh(core_axis_name="c", subcore_axis_name="s")


def _zero_fill(ref, nvec):
    z = jnp.zeros((16,), jnp.float32)

    def body(i, _):
        ref[pl.ds(i * 16, 16)] = z
        return 0

    lax.fori_loop(0, nvec, body, 0)


def _zero_fill2d(ref):
    z = jnp.zeros((16,), jnp.float32)

    def body(i, _):
        for k in range(8):
            ref[i, pl.ds(k * 16, 16)] = z
        return 0

    lax.fori_loop(0, ref.shape[0], body, 0)


def _prep_body(esrc, edst, et, norm_out, gidx_out, cnt_sh, zbuf, ones,
               sbig, dbig, tbig, kbig, g0big, g1big, nbig, kidx_a, kidx_b,
               cbuf, ssem_a, ssem_b):
    c = lax.axis_index("c")
    s = lax.axis_index("s")

    @pl.when(c == 0)
    def _():
        _zero_fill(zbuf, 3760 // 16)
        pltpu.sync_copy(zbuf, cnt_sh.at[pl.ds(s * 3760, 3760)])

        def init_ones(i, _):
            ones[pl.ds(i * 16, 16)] = jnp.full((16,), 1.0, jnp.float32)
            return 0

        lax.fori_loop(0, _SUB // 16, init_ones, 0)
        plsc.subcore_barrier()

        e0 = s * _EPT

        def count_super(i, _):
            base = e0 + i * _EPS
            pltpu.sync_copy(esrc.at[pl.ds(base, _EPS)], sbig)
            pltpu.sync_copy(edst.at[pl.ds(base, _EPS)], dbig)
            pltpu.sync_copy(et.at[pl.ds(base, _EPS)], tbig)

            def vec(j, _):
                dv = dbig[pl.ds(j * 16, 16)]
                tv = tbig[pl.ds(j * 16, 16)]
                sv = sbig[pl.ds(j * 16, 16)]
                kbig[pl.ds(j * 16, 16)] = dv * _R + tv
                g0 = tv * _N + sv
                g0big[pl.ds(j * 16, 16)] = g0
                g1big[pl.ds(j * 16, 16)] = g0 + _R * _N
                return 0

            lax.fori_loop(0, _EPS // 16, vec, 0)
            pltpu.sync_copy(g0big, gidx_out.at[pl.ds(base, _EPS)])
            pltpu.sync_copy(g1big, gidx_out.at[pl.ds(_E + base, _EPS)])

            kbufs = (kidx_a, kidx_b)
            sems = (ssem_a, ssem_b)
            sdesc = [None] * _NSUB
            for j in range(_NSUB):
                kb = kbufs[j % 2]
                if j >= 2:
                    sdesc[j - 2].wait()
                for k in range(_SUB // 16):
                    kb[pl.ds(k * 16, 16)] = kbig[pl.ds(j * _SUB + k * 16, 16)]
                sdesc[j] = pltpu.async_copy(ones, cnt_sh.at[kb], sems[j % 2],
                                            add=True)
            sdesc[_NSUB - 2].wait()
            sdesc[_NSUB - 1].wait()
            return 0

        lax.fori_loop(0, _EPT // _EPS, count_super, 0)
        plsc.subcore_barrier()

        pltpu.sync_copy(cnt_sh, cbuf)

        def norm_super(i, _):
            base = e0 + i * _EPS
            pltpu.sync_copy(edst.at[pl.ds(base, _EPS)], dbig)
            pltpu.sync_copy(et.at[pl.ds(base, _EPS)], tbig)

            def vec(j, _):
                dv = dbig[pl.ds(j * 16, 16)]
                tv = tbig[pl.ds(j * 16, 16)]
                cv = plsc.load_gather(cbuf, [dv * _R + tv])
                nbig[pl.ds(j * 16, 16)] = 1.0 / jnp.maximum(cv, 1.0)
                return 0

            lax.fori_loop(0, _EPS // 16, vec, 0)
            pltpu.sync_copy(nbig, norm_out.at[pl.ds(base, _EPS)])
            return 0

        lax.fori_loop(0, _EPT // _EPS, norm_super, 0)


def _sc_prep(esrc, edst, edge_types):
    f = pl.kernel(
        _prep_body,
        out_type=(
            jax.ShapeDtypeStruct((_E,), jnp.float32),
            jax.ShapeDtypeStruct((2 * _E,), jnp.int32),
        ),
        mesh=_sc_mesh(),
        scratch_types=[
            pltpu.MemorySpace.VMEM_SHARED((_NR_PAD,), jnp.float32),
            pltpu.VMEM((3760,), jnp.float32),
            pltpu.VMEM((_SUB,), jnp.float32),
            pltpu.VMEM((_EPS,), jnp.int32),
            pltpu.VMEM((_EPS,), jnp.int32),
            pltpu.VMEM((_EPS,), jnp.int32),
            pltpu.VMEM((_EPS,), jnp.int32),
            pltpu.VMEM((_EPS,), jnp.int32),
            pltpu.VMEM((_EPS,), jnp.int32),
            pltpu.VMEM((_EPS,), jnp.float32),
            pltpu.VMEM((_SUB,), jnp.int32),
            pltpu.VMEM((_SUB,), jnp.int32),
            pltpu.VMEM((_NR_PAD,), jnp.float32),
            pltpu.SemaphoreType.DMA,
            pltpu.SemaphoreType.DMA,
        ],
        compiler_params=pltpu.CompilerParams(needs_layout_passes=False),
    )
    return f(esrc, edst, edge_types)


def _agg_body(xw, gidx2, edst, norm, out, agg_sh, zbuf, gbig, dbig, nbig,
              rows0, rows1, rows2, rows3, didx0, didx1, didx2, didx3,
              gidx0, gidx1, gidx2b, gidx3, gsem0, gsem1, gsem2, gsem3,
              ssem0, ssem1, ssem2, ssem3):
    c = lax.axis_index("c")
    s = lax.axis_index("s")

    _zero_fill2d(zbuf)
    z0 = s * 624

    def zrow(k, _):
        pltpu.sync_copy(zbuf, agg_sh.at[pl.ds(z0 + k * 16, 16)])
        return 0

    lax.fori_loop(0, 39, zrow, 0)

    @pl.when(s == 0)
    def _():
        pltpu.sync_copy(zbuf, agg_sh.at[pl.ds(9984, 16)])

    plsc.subcore_barrier()

    e0 = s * _EPT
    rows = (rows0, rows1, rows2, rows3)
    didx = (didx0, didx1, didx2, didx3)
    gidx = (gidx0, gidx1, gidx2b, gidx3)
    gsem = (gsem0, gsem1, gsem2, gsem3)
    ssem = (ssem0, ssem1, ssem2, ssem3)

    def scale(rbuf, joff):
        base = jnp.full((16,), joff, jnp.int32)

        def body(t, _):
            e = t * 2
            for d in range(2):
                ns = plsc.load_gather(nbig, [base + (e + d)])
                for k in range(8):
                    rbuf[e + d, pl.ds(k * 16, 16)] = (
                        rbuf[e + d, pl.ds(k * 16, 16)] * ns)
            return 0

        lax.fori_loop(0, _SUB // 2, body, 0)

    def fill_idx(dst_ref, src_ref, joff):
        for k in range(_SUB // 16):
            dst_ref[pl.ds(k * 16, 16)] = src_ref[pl.ds(joff + k * 16, 16)]

    def super_chunk(i, _):
        base = e0 + i * _EPS
        pltpu.sync_copy(gidx2.at[pl.ds(c * _E + base, _EPS)], gbig)
        pltpu.sync_copy(edst.at[pl.ds(base, _EPS)], dbig)
        pltpu.sync_copy(norm.at[pl.ds(base, _EPS)], nbig)

        gdesc = [None] * _NSUB
        sdesc = [None] * _NSUB
        for p in range(3):
            b = p % 4
            fill_idx(gidx[b], gbig, p * _SUB)
            gdesc[p] = pltpu.async_copy(xw.at[gidx[b]], rows[b], gsem[b])
        for j in range(_NSUB):
            b = j % 4
            gdesc[j].wait()
            fill_idx(didx[b], dbig, j * _SUB)
            scale(rows[b], j * _SUB)
            sdesc[j] = pltpu.async_copy(rows[b], agg_sh.at[didx[b]], ssem[b],
                                        add=True)
            jn = j + 3
            if jn < _NSUB:
                bn = jn % 4
                if jn - 4 >= 0:
                    sdesc[jn - 4].wait()
                fill_idx(gidx[bn], gbig, jn * _SUB)
                gdesc[jn] = pltpu.async_copy(xw.at[gidx[bn]], rows[bn],
                                             gsem[bn])
        for j in range(max(0, _NSUB - 4), _NSUB):
            sdesc[j].wait()
        return 0

    lax.fori_loop(0, _EPT // _EPS, super_chunk, 0)
    plsc.subcore_barrier()

    @pl.when(s == 0)
    def _():
        pltpu.sync_copy(agg_sh, out.at[c])


def _sc_aggregate(xw2, gidx2, edst, norm):
    f = pl.kernel(
        _agg_body,
        out_type=jax.ShapeDtypeStruct((2, _N, 128), jnp.float32),
        mesh=_sc_mesh(),
        scratch_types=(
            [pltpu.MemorySpace.VMEM_SHARED((_N, 128), jnp.float32),
             pltpu.VMEM((16, 128), jnp.float32),
             pltpu.VMEM((_EPS,), jnp.int32),
             pltpu.VMEM((_EPS,), jnp.int32),
             pltpu.VMEM((_EPS,), jnp.float32)]
            + [pltpu.VMEM((_SUB, 128), jnp.float32)] * 4
            + [pltpu.VMEM((_SUB,), jnp.int32)] * 4
            + [pltpu.VMEM((_SUB,), jnp.int32)] * 4
            + [pltpu.SemaphoreType.DMA] * 8
        ),
        compiler_params=pltpu.CompilerParams(needs_layout_passes=False),
    )
    return f(xw2, gidx2, edst, norm)


def _inproj_body(h_ref, w_ref, b_ref, o_ref):
    y = jnp.dot(h_ref[...], w_ref[...], preferred_element_type=jnp.float32)
    o_ref[...] = jax.nn.relu(y + b_ref[...])


def _input_proj(h_text, W_in, b_in):
    n, hid = h_text.shape
    d = W_in.shape[1]
    return pl.pallas_call(
        _inproj_body,
        grid=(n // _BN,),
        in_specs=[
            pl.BlockSpec((_BN, hid), lambda i: (i, 0)),
            pl.BlockSpec((hid, d), lambda i: (0, 0)),
            pl.BlockSpec((1, d), lambda i: (0, 0)),
        ],
        out_specs=pl.BlockSpec((_BN, d), lambda i: (i, 0)),
        out_shape=jax.ShapeDtypeStruct((n, d), jnp.float32),
    )(h_text, W_in, b_in.reshape(1, d))


def _wcomb_body(comp_ref, bases_ref, o_ref):
    r, b = comp_ref.shape[1], comp_ref.shape[2]
    for ri in range(r):
        acc = comp_ref[0, ri, 0] * bases_ref[0, 0]
        for i in range(1, b):
            acc = acc + comp_ref[0, ri, i] * bases_ref[0, i]
        o_ref[0, ri] = acc


def _basis_combine(comp, bases):
    ll, r, b = comp.shape
    d = bases.shape[-1]
    return pl.pallas_call(
        _wcomb_body,
        grid=(ll,),
        in_specs=[
            pl.BlockSpec((1, r, b), lambda l: (l, 0, 0)),
            pl.BlockSpec((1, b, d, d), lambda l: (l, 0, 0, 0)),
        ],
        out_specs=pl.BlockSpec((1, r, d, d), lambda l: (l, 0, 0, 0)),
        out_shape=jax.ShapeDtypeStruct((ll, r, d, d), jnp.float32),
    )(comp, bases)


def _xw_body(x_ref, w_ref, o_ref):
    y = jnp.dot(x_ref[...], w_ref[0], preferred_element_type=jnp.float32)
    h = y.shape[-1] // 2
    o_ref[0, 0] = y[:, :h]
    o_ref[1, 0] = y[:, h:]


def _per_relation_matmul(x, W):
    n, d = x.shape
    r = W.shape[0]
    return pl.pallas_call(
        _xw_body,
        grid=(r, n // _BN),
        in_specs=[
            pl.BlockSpec((_BN, d), lambda ri, i: (i, 0)),
            pl.BlockSpec((1, d, d), lambda ri, i: (ri, 0, 0)),
        ],
        out_specs=pl.BlockSpec((2, 1, _BN, d // 2),
                               lambda ri, i: (0, ri, i, 0)),
        out_shape=jax.ShapeDtypeStruct((2, r, n, d // 2), jnp.float32),
    )(x, W)


def _post_body(x_ref, agg_ref, root_ref, rb_ref, g_ref, be_ref, o_ref):
    agg = jnp.concatenate([agg_ref[0], agg_ref[1]], axis=-1)
    y = agg + jnp.dot(x_ref[...], root_ref[...],
                      preferred_element_type=jnp.float32) + rb_ref[...]
    y = jax.nn.relu(y) + x_ref[...]
    scale = 1.0 / jnp.sqrt(1.0 + 1e-5)
    o_ref[...] = y * (g_ref[...] * scale) + be_ref[...]


def _layer_post(x, agg2, root, rbias, gamma, beta):
    n, d = x.shape
    return pl.pallas_call(
        _post_body,
        grid=(n // _BN,),
        in_specs=[
            pl.BlockSpec((_BN, d), lambda i: (i, 0)),
            pl.BlockSpec((2, _BN, d // 2), lambda i: (0, i, 0)),
            pl.BlockSpec((d, d), lambda i: (0, 0)),
            pl.BlockSpec((1, d), lambda i: (0, 0)),
            pl.BlockSpec((1, d), lambda i: (0, 0)),
            pl.BlockSpec((1, d), lambda i: (0, 0)),
        ],
        out_specs=pl.BlockSpec((_BN, d), lambda i: (i, 0)),
        out_shape=jax.ShapeDtypeStruct((n, d), jnp.float32),
    )(x, agg2, root, rbias.reshape(1, d), gamma.reshape(1, d),
      beta.reshape(1, d))


def _out_body(x_ref, w_ref, b_ref, g_ref, be_ref, o_ref):
    h = jnp.dot(x_ref[...], w_ref[...], preferred_element_type=jnp.float32)
    h = h + b_ref[...]
    mu = jnp.mean(h, axis=-1, keepdims=True)
    var = jnp.mean((h - mu) ** 2, axis=-1, keepdims=True)
    o_ref[...] = (h - mu) / jnp.sqrt(var + 1e-5) * g_ref[...] + be_ref[...]


def _output_proj(x, W_out, b_out, ln_gamma, ln_beta):
    n, d = x.shape
    hid = W_out.shape[1]
    return pl.pallas_call(
        _out_body,
        grid=(n // _BN,),
        in_specs=[
            pl.BlockSpec((_BN, d), lambda i: (i, 0)),
            pl.BlockSpec((d, hid), lambda i: (0, 0)),
            pl.BlockSpec((1, hid), lambda i: (0, 0)),
            pl.BlockSpec((1, hid), lambda i: (0, 0)),
            pl.BlockSpec((1, hid), lambda i: (0, 0)),
        ],
        out_specs=pl.BlockSpec((_BN, hid), lambda i: (i, 0)),
        out_shape=jax.ShapeDtypeStruct((n, hid), jnp.float32),
    )(x, W_out, b_out.reshape(1, hid), ln_gamma.reshape(1, hid),
      ln_beta.reshape(1, hid))


def kernel(h_text, edge_index, edge_types, W_in, b_in, bases, comp, root,
           rbias, bn_gamma, bn_beta, W_out, b_out, ln_gamma, ln_beta):
    num_l = comp.shape[0]

    x = _input_proj(h_text, W_in, b_in)
    W_all = _basis_combine(comp, bases)
    esrc = edge_index[0]
    edst = edge_index[1]
    norm, gidx2 = _sc_prep(esrc, edst, edge_types)

    for l in range(num_l):
        xw2 = _per_relation_matmul(x, W_all[l])
        xw2 = xw2.reshape(2 * _R * _N, 128)
        agg2 = _sc_aggregate(xw2, gidx2, edst, norm)
        x = _layer_post(x, agg2, root[l], rbias[l], bn_gamma[l], bn_beta[l])

    return _output_proj(x, W_out, b_out, ln_gamma, ln_beta)


# speedup vs baseline: 1.5351x; 1.1919x over previous
import jax
import jax.numpy as jnp
from jax import lax
from jax.experimental import pallas as pl
from jax.experimental.pallas import tpu as pltpu
from jax.experimental.pallas import tpu_sc as plsc

_BN = 1000
_SUB = 80
_EPS = 800
_BIG = 2000
_NSUB = _EPS // _SUB

_N = 10000
_E = 320000
_R = 6
_NR_PAD = 60160
_EPT = _E // 16


def _sc_mesh():
    return plsc.VectorSubcoreMesh(core_axis_name="c", subcore_axis_name="s")


def _zero_fill(ref, nvec):
    z = jnp.zeros((16,), jnp.float32)

    def body(i, _):
        ref[pl.ds(i * 16, 16)] = z
        return 0

    lax.fori_loop(0, nvec, body, 0)


def _zero_fill2d(ref):
    z = jnp.zeros((16,), jnp.float32)

    def body(i, _):
        for k in range(8):
            ref[i, pl.ds(k * 16, 16)] = z
        return 0

    lax.fori_loop(0, ref.shape[0], body, 0)


def _prep_body(esrc, edst, et, norm_out, gidx_out, cnt_sh, zbuf, ones,
               sbig, dbig, tbig, kbig, g0big, g1big, nbig, kidx_a, kidx_b,
               cbuf, ssem_a, ssem_b):
    c = lax.axis_index("c")
    s = lax.axis_index("s")

    @pl.when(c == 0)
    def _():
        _zero_fill(zbuf, 3760 // 16)
        pltpu.sync_copy(zbuf, cnt_sh.at[pl.ds(s * 3760, 3760)])

        def init_ones(i, _):
            ones[pl.ds(i * 16, 16)] = jnp.full((16,), 1.0, jnp.float32)
            return 0

        lax.fori_loop(0, _SUB // 16, init_ones, 0)
        plsc.subcore_barrier()

        e0 = s * _EPT

        def count_super(i, _):
            base = e0 + i * _EPS
            pltpu.sync_copy(esrc.at[pl.ds(base, _EPS)], sbig)
            pltpu.sync_copy(edst.at[pl.ds(base, _EPS)], dbig)
            pltpu.sync_copy(et.at[pl.ds(base, _EPS)], tbig)

            def vec(j, _):
                dv = dbig[pl.ds(j * 16, 16)]
                tv = tbig[pl.ds(j * 16, 16)]
                sv = sbig[pl.ds(j * 16, 16)]
                kbig[pl.ds(j * 16, 16)] = dv * _R + tv
                g0 = tv * _N + sv
                g0big[pl.ds(j * 16, 16)] = g0
                g1big[pl.ds(j * 16, 16)] = g0 + _R * _N
                return 0

            lax.fori_loop(0, _EPS // 16, vec, 0)
            pltpu.sync_copy(g0big, gidx_out.at[pl.ds(base, _EPS)])
            pltpu.sync_copy(g1big, gidx_out.at[pl.ds(_E + base, _EPS)])

            kbufs = (kidx_a, kidx_b)
            sems = (ssem_a, ssem_b)
            sdesc = [None] * _NSUB
            for j in range(_NSUB):
                kb = kbufs[j % 2]
                if j >= 2:
                    sdesc[j - 2].wait()
                for k in range(_SUB // 16):
                    kb[pl.ds(k * 16, 16)] = kbig[pl.ds(j * _SUB + k * 16, 16)]
                sdesc[j] = pltpu.async_copy(ones, cnt_sh.at[kb], sems[j % 2],
                                            add=True)
            sdesc[_NSUB - 2].wait()
            sdesc[_NSUB - 1].wait()
            return 0

        lax.fori_loop(0, _EPT // _EPS, count_super, 0)
        plsc.subcore_barrier()

        pltpu.sync_copy(cnt_sh, cbuf)

        def norm_super(i, _):
            base = e0 + i * _EPS
            pltpu.sync_copy(edst.at[pl.ds(base, _EPS)], dbig)
            pltpu.sync_copy(et.at[pl.ds(base, _EPS)], tbig)

            def vec(j, _):
                dv = dbig[pl.ds(j * 16, 16)]
                tv = tbig[pl.ds(j * 16, 16)]
                cv = plsc.load_gather(cbuf, [dv * _R + tv])
                nbig[pl.ds(j * 16, 16)] = 1.0 / jnp.maximum(cv, 1.0)
                return 0

            lax.fori_loop(0, _EPS // 16, vec, 0)
            pltpu.sync_copy(nbig, norm_out.at[pl.ds(base, _EPS)])
            return 0

        lax.fori_loop(0, _EPT // _EPS, norm_super, 0)


def _sc_prep(esrc, edst, edge_types):
    f = pl.kernel(
        _prep_body,
        out_type=(
            jax.ShapeDtypeStruct((_E,), jnp.float32),
            jax.ShapeDtypeStruct((2 * _E,), jnp.int32),
        ),
        mesh=_sc_mesh(),
        scratch_types=[
            pltpu.MemorySpace.VMEM_SHARED((_NR_PAD,), jnp.float32),
            pltpu.VMEM((3760,), jnp.float32),
            pltpu.VMEM((_SUB,), jnp.float32),
            pltpu.VMEM((_EPS,), jnp.int32),
            pltpu.VMEM((_EPS,), jnp.int32),
            pltpu.VMEM((_EPS,), jnp.int32),
            pltpu.VMEM((_EPS,), jnp.int32),
            pltpu.VMEM((_EPS,), jnp.int32),
            pltpu.VMEM((_EPS,), jnp.int32),
            pltpu.VMEM((_EPS,), jnp.float32),
            pltpu.VMEM((_SUB,), jnp.int32),
            pltpu.VMEM((_SUB,), jnp.int32),
            pltpu.VMEM((_NR_PAD,), jnp.float32),
            pltpu.SemaphoreType.DMA,
            pltpu.SemaphoreType.DMA,
        ],
        compiler_params=pltpu.CompilerParams(needs_layout_passes=False),
    )
    return f(esrc, edst, edge_types)


def _agg_body(xw, gidx2, edst, norm, out, agg_sh, zbuf, gbig, dbig, nbig,
              rows0, rows1, rows2, rows3, didx0, didx1, didx2, didx3,
              gidx0, gidx1, gidx2b, gidx3, gsem0, gsem1, gsem2, gsem3,
              ssem0, ssem1, ssem2, ssem3):
    c = lax.axis_index("c")
    s = lax.axis_index("s")

    _zero_fill2d(zbuf)
    z0 = s * 624

    def zrow(k, _):
        pltpu.sync_copy(zbuf, agg_sh.at[pl.ds(z0 + k * 16, 16)])
        return 0

    lax.fori_loop(0, 39, zrow, 0)

    @pl.when(s == 0)
    def _():
        pltpu.sync_copy(zbuf, agg_sh.at[pl.ds(9984, 16)])

    plsc.subcore_barrier()

    e0 = s * _EPT
    rows = (rows0, rows1, rows2, rows3)
    didx = (didx0, didx1, didx2, didx3)
    gidx = (gidx0, gidx1, gidx2b, gidx3)
    gsem = (gsem0, gsem1, gsem2, gsem3)
    ssem = (ssem0, ssem1, ssem2, ssem3)

    def scale(rbuf, joff):
        base = jnp.full((16,), joff, jnp.int32)

        def body(t, _):
            e = t * 2
            for d in range(2):
                ns = plsc.load_gather(nbig, [base + (e + d)])
                for k in range(8):
                    rbuf[e + d, pl.ds(k * 16, 16)] = (
                        rbuf[e + d, pl.ds(k * 16, 16)] * ns)
            return 0

        lax.fori_loop(0, _SUB // 2, body, 0)

    def fill_idx(dst_ref, src_ref, joff):
        for k in range(_SUB // 16):
            dst_ref[pl.ds(k * 16, 16)] = src_ref[pl.ds(joff + k * 16, 16)]

    def stage_block(o, _):
        sbase = e0 + o * _BIG
        pltpu.sync_copy(gidx2.at[pl.ds(c * _E + sbase, _BIG)], gbig)
        pltpu.sync_copy(edst.at[pl.ds(sbase, _BIG)], dbig)
        pltpu.sync_copy(norm.at[pl.ds(sbase, _BIG)], nbig)

        def super_chunk(i, _):
            off = i * _EPS
            gdesc = [None] * _NSUB
            sdesc = [None] * _NSUB
            for p in range(3):
                b = p % 4
                fill_idx(gidx[b], gbig, off + p * _SUB)
                gdesc[p] = pltpu.async_copy(xw.at[gidx[b]], rows[b], gsem[b])
            for j in range(_NSUB):
                b = j % 4
                gdesc[j].wait()
                fill_idx(didx[b], dbig, off + j * _SUB)
                scale(rows[b], off + j * _SUB)
                sdesc[j] = pltpu.async_copy(rows[b], agg_sh.at[didx[b]],
                                            ssem[b], add=True)
                jn = j + 3
                if jn < _NSUB:
                    bn = jn % 4
                    if jn - 4 >= 0:
                        sdesc[jn - 4].wait()
                    fill_idx(gidx[bn], gbig, off + jn * _SUB)
                    gdesc[jn] = pltpu.async_copy(xw.at[gidx[bn]], rows[bn],
                                                 gsem[bn])
            for j in range(max(0, _NSUB - 4), _NSUB):
                sdesc[j].wait()
            return 0

        lax.fori_loop(0, _BIG // _EPS, super_chunk, 0)
        return 0

    lax.fori_loop(0, _EPT // _BIG, stage_block, 0)
    plsc.subcore_barrier()

    @pl.when(s == 0)
    def _():
        pltpu.sync_copy(agg_sh, out.at[c])


def _sc_aggregate(xw2, gidx2, edst, norm):
    f = pl.kernel(
        _agg_body,
        out_type=jax.ShapeDtypeStruct((2, _N, 128), jnp.float32),
        mesh=_sc_mesh(),
        scratch_types=(
            [pltpu.MemorySpace.VMEM_SHARED((_N, 128), jnp.float32),
             pltpu.VMEM((16, 128), jnp.float32),
             pltpu.VMEM((_BIG,), jnp.int32),
             pltpu.VMEM((_BIG,), jnp.int32),
             pltpu.VMEM((_BIG,), jnp.float32)]
            + [pltpu.VMEM((_SUB, 128), jnp.float32)] * 4
            + [pltpu.VMEM((_SUB,), jnp.int32)] * 4
            + [pltpu.VMEM((_SUB,), jnp.int32)] * 4
            + [pltpu.SemaphoreType.DMA] * 8
        ),
        compiler_params=pltpu.CompilerParams(needs_layout_passes=False),
    )
    return f(xw2, gidx2, edst, norm)


def _inproj_body(h_ref, w_ref, b_ref, o_ref):
    y = jnp.dot(h_ref[...], w_ref[...], preferred_element_type=jnp.float32)
    o_ref[...] = jax.nn.relu(y + b_ref[...])


def _input_proj(h_text, W_in, b_in):
    n, hid = h_text.shape
    d = W_in.shape[1]
    return pl.pallas_call(
        _inproj_body,
        grid=(n // _BN,),
        in_specs=[
            pl.BlockSpec((_BN, hid), lambda i: (i, 0)),
            pl.BlockSpec((hid, d), lambda i: (0, 0)),
            pl.BlockSpec((1, d), lambda i: (0, 0)),
        ],
        out_specs=pl.BlockSpec((_BN, d), lambda i: (i, 0)),
        out_shape=jax.ShapeDtypeStruct((n, d), jnp.float32),
    )(h_text, W_in, b_in.reshape(1, d))


def _wcomb_body(comp_ref, bases_ref, o_ref):
    r, b = comp_ref.shape[1], comp_ref.shape[2]
    for ri in range(r):
        acc = comp_ref[0, ri, 0] * bases_ref[0, 0]
        for i in range(1, b):
            acc = acc + comp_ref[0, ri, i] * bases_ref[0, i]
        o_ref[0, ri] = acc


def _basis_combine(comp, bases):
    ll, r, b = comp.shape
    d = bases.shape[-1]
    return pl.pallas_call(
        _wcomb_body,
        grid=(ll,),
        in_specs=[
            pl.BlockSpec((1, r, b), lambda l: (l, 0, 0)),
            pl.BlockSpec((1, b, d, d), lambda l: (l, 0, 0, 0)),
        ],
        out_specs=pl.BlockSpec((1, r, d, d), lambda l: (l, 0, 0, 0)),
        out_shape=jax.ShapeDtypeStruct((ll, r, d, d), jnp.float32),
    )(comp, bases)


def _xw_body(x_ref, w_ref, o_ref):
    y = jnp.dot(x_ref[...], w_ref[0], preferred_element_type=jnp.float32)
    h = y.shape[-1] // 2
    o_ref[0, 0] = y[:, :h]
    o_ref[1, 0] = y[:, h:]


def _per_relation_matmul(x, W):
    n, d = x.shape
    r = W.shape[0]
    return pl.pallas_call(
        _xw_body,
        grid=(r, n // _BN),
        in_specs=[
            pl.BlockSpec((_BN, d), lambda ri, i: (i, 0)),
            pl.BlockSpec((1, d, d), lambda ri, i: (ri, 0, 0)),
        ],
        out_specs=pl.BlockSpec((2, 1, _BN, d // 2),
                               lambda ri, i: (0, ri, i, 0)),
        out_shape=jax.ShapeDtypeStruct((2, r, n, d // 2), jnp.float32),
    )(x, W)


def _post_body(x_ref, agg_ref, root_ref, rb_ref, g_ref, be_ref, o_ref):
    agg = jnp.concatenate([agg_ref[0], agg_ref[1]], axis=-1)
    y = agg + jnp.dot(x_ref[...], root_ref[...],
                      preferred_element_type=jnp.float32) + rb_ref[...]
    y = jax.nn.relu(y) + x_ref[...]
    scale = 1.0 / jnp.sqrt(1.0 + 1e-5)
    o_ref[...] = y * (g_ref[...] * scale) + be_ref[...]


def _layer_post(x, agg2, root, rbias, gamma, beta):
    n, d = x.shape
    return pl.pallas_call(
        _post_body,
        grid=(n // _BN,),
        in_specs=[
            pl.BlockSpec((_BN, d), lambda i: (i, 0)),
            pl.BlockSpec((2, _BN, d // 2), lambda i: (0, i, 0)),
            pl.BlockSpec((d, d), lambda i: (0, 0)),
            pl.BlockSpec((1, d), lambda i: (0, 0)),
            pl.BlockSpec((1, d), lambda i: (0, 0)),
            pl.BlockSpec((1, d), lambda i: (0, 0)),
        ],
        out_specs=pl.BlockSpec((_BN, d), lambda i: (i, 0)),
        out_shape=jax.ShapeDtypeStruct((n, d), jnp.float32),
    )(x, agg2, root, rbias.reshape(1, d), gamma.reshape(1, d),
      beta.reshape(1, d))


def _out_body(x_ref, w_ref, b_ref, g_ref, be_ref, o_ref):
    h = jnp.dot(x_ref[...], w_ref[...], preferred_element_type=jnp.float32)
    h = h + b_ref[...]
    mu = jnp.mean(h, axis=-1, keepdims=True)
    var = jnp.mean((h - mu) ** 2, axis=-1, keepdims=True)
    o_ref[...] = (h - mu) / jnp.sqrt(var + 1e-5) * g_ref[...] + be_ref[...]


def _output_proj(x, W_out, b_out, ln_gamma, ln_beta):
    n, d = x.shape
    hid = W_out.shape[1]
    return pl.pallas_call(
        _out_body,
        grid=(n // _BN,),
        in_specs=[
            pl.BlockSpec((_BN, d), lambda i: (i, 0)),
            pl.BlockSpec((d, hid), lambda i: (0, 0)),
            pl.BlockSpec((1, hid), lambda i: (0, 0)),
            pl.BlockSpec((1, hid), lambda i: (0, 0)),
            pl.BlockSpec((1, hid), lambda i: (0, 0)),
        ],
        out_specs=pl.BlockSpec((_BN, hid), lambda i: (i, 0)),
        out_shape=jax.ShapeDtypeStruct((n, hid), jnp.float32),
    )(x, W_out, b_out.reshape(1, hid), ln_gamma.reshape(1, hid),
      ln_beta.reshape(1, hid))


def kernel(h_text, edge_index, edge_types, W_in, b_in, bases, comp, root,
           rbias, bn_gamma, bn_beta, W_out, b_out, ln_gamma, ln_beta):
    num_l = comp.shape[0]

    x = _input_proj(h_text, W_in, b_in)
    W_all = _basis_combine(comp, bases)
    esrc = edge_index[0]
    edst = edge_index[1]
    norm, gidx2 = _sc_prep(esrc, edst, edge_types)

    for l in range(num_l):
        xw2 = _per_relation_matmul(x, W_all[l])
        xw2 = xw2.reshape(2 * _R * _N, 128)
        agg2 = _sc_aggregate(xw2, gidx2, edst, norm)
        x = _layer_post(x, agg2, root[l], rbias[l], bn_gamma[l], bn_beta[l])

    return _output_proj(x, W_out, b_out, ln_gamma, ln_beta)
